# baseline (speedup 1.0000x reference)
import math

import jax
import jax.numpy as jnp
from jax.experimental import pallas as pl
from jax.experimental.pallas import tpu as pltpu

N = 131072
COLS = 128
ROWS = N // COLS
BLK = 512
GRID = ROWS // BLK

_LOG_HALF = math.log(0.5)
_LOG_2PI = math.log(2.0 * math.pi)
_OBS_STD = 0.1
_K_OBS = 0.5 / (_OBS_STD * _OBS_STD)
_K_1 = 0.5
_K_20 = 0.5
_K_21 = 0.5 / (1.5 * 1.5)
_C_1 = _LOG_HALF - math.log(1.0) - math.log(_OBS_STD) - _LOG_2PI
_C_20 = 2.0 * _LOG_HALF - math.log(1.0) - math.log(_OBS_STD) - _LOG_2PI
_C_21 = 2.0 * _LOG_HALF - math.log(1.5) - math.log(_OBS_STD) - _LOG_2PI

_MEAN_20 = -2.0
_MEAN_21 = 2.0

_f32 = jnp.float32


def _logpdf_body(mean_ref, x1, o1, x20, o20, x21, o21, y1, y20, y21):
    m = mean_ref[0, 0]

    x = x1[...]
    o = o1[...]
    d = x - m
    e = o - x
    y1[...] = _C_1 - _K_1 * (d * d) - _K_OBS * (e * e)

    x = x20[...]
    o = o20[...]
    d = x - _MEAN_20
    e = o - x
    y20[...] = _C_20 - _K_20 * (d * d) - _K_OBS * (e * e)

    x = x21[...]
    o = o21[...]
    d = x - _MEAN_21
    e = o - x
    y21[...] = _C_21 - _K_21 * (d * d) - _K_OBS * (e * e)


_block = pl.BlockSpec((BLK, COLS), lambda i: (i, 0))

_logpdf_call = pl.pallas_call(
    _logpdf_body,
    grid=(GRID,),
    in_specs=[
        pl.BlockSpec(memory_space=pltpu.SMEM),
        _block, _block, _block, _block, _block, _block,
    ],
    out_specs=(_block, _block, _block),
    out_shape=(
        jax.ShapeDtypeStruct((ROWS, COLS), _f32),
        jax.ShapeDtypeStruct((ROWS, COLS), _f32),
        jax.ShapeDtypeStruct((ROWS, COLS), _f32),
    ),
)


def kernel(k_1, x_1, obs_1, k_20, z_0, x_20, obs_20, k_21, z_1, x_21, obs_21,
           mean_1):
    del k_1, k_20, z_0, k_21, z_1
    mean_11 = mean_1.astype(_f32).reshape(1, 1)
    args = [a.reshape(ROWS, COLS)
            for a in (x_1, obs_1, x_20, obs_20, x_21, obs_21)]
    y1, y20, y21 = _logpdf_call(mean_11, *args)
    return (y1.reshape(N), y20.reshape(N), y21.reshape(N))


# speedup vs baseline: 1.0030x; 1.0030x over previous
import math

import jax
import jax.numpy as jnp
from jax.experimental import pallas as pl
from jax.experimental.pallas import tpu as pltpu

N = 131072
COLS = 128
ROWS = N // COLS
BLK = 512
GRID = ROWS // BLK

_LOG_HALF = math.log(0.5)
_LOG_2PI = math.log(2.0 * math.pi)
_OBS_STD = 0.1
_K_OBS = 0.5 / (_OBS_STD * _OBS_STD)
_K_1 = 0.5
_K_20 = 0.5
_K_21 = 0.5 / (1.5 * 1.5)
_C_1 = _LOG_HALF - math.log(1.0) - math.log(_OBS_STD) - _LOG_2PI
_C_20 = 2.0 * _LOG_HALF - math.log(1.0) - math.log(_OBS_STD) - _LOG_2PI
_C_21 = 2.0 * _LOG_HALF - math.log(1.5) - math.log(_OBS_STD) - _LOG_2PI

_MEAN_20 = -2.0
_MEAN_21 = 2.0

_f32 = jnp.float32


def _logpdf_body(mean_ref, x1, o1, x20, o20, x21, o21, y1, y20, y21):
    m = mean_ref[0, 0]

    x = x1[...]
    o = o1[...]
    d = x - m
    e = o - x
    y1[...] = _C_1 - _K_1 * (d * d) - _K_OBS * (e * e)

    x = x20[...]
    o = o20[...]
    d = x - _MEAN_20
    e = o - x
    y20[...] = _C_20 - _K_20 * (d * d) - _K_OBS * (e * e)

    x = x21[...]
    o = o21[...]
    d = x - _MEAN_21
    e = o - x
    y21[...] = _C_21 - _K_21 * (d * d) - _K_OBS * (e * e)


_block = pl.BlockSpec((BLK, COLS), lambda i: (i, 0))

_logpdf_call = pl.pallas_call(
    _logpdf_body,
    grid=(GRID,),
    in_specs=[
        pl.BlockSpec(memory_space=pltpu.SMEM),
        _block, _block, _block, _block, _block, _block,
    ],
    out_specs=(_block, _block, _block),
    out_shape=(
        jax.ShapeDtypeStruct((ROWS, COLS), _f32),
        jax.ShapeDtypeStruct((ROWS, COLS), _f32),
        jax.ShapeDtypeStruct((ROWS, COLS), _f32),
    ),
    compiler_params=pltpu.CompilerParams(
        dimension_semantics=("parallel",),
    ),
)


def kernel(k_1, x_1, obs_1, k_20, z_0, x_20, obs_20, k_21, z_1, x_21, obs_21,
           mean_1):
    del k_1, k_20, z_0, k_21, z_1
    mean_11 = mean_1.astype(_f32).reshape(1, 1)
    args = [a.reshape(ROWS, COLS)
            for a in (x_1, obs_1, x_20, obs_20, x_21, obs_21)]
    y1, y20, y21 = _logpdf_call(mean_11, *args)
    return (y1.reshape(N), y20.reshape(N), y21.reshape(N))
